# initial kernel scaffold (unmeasured)
import jax
import jax.numpy as jnp
from jax import lax
from jax.experimental import pallas as pl
from jax.experimental.pallas import tpu as pltpu

N_DEV = 8
T = 2048
D = 1024
SLOT = 512
ALIGN = 16


def _a2a_body(xs_ref, dest_ref, stage_ref, dall_ref,
              local_sems, send_data_sems, send_dest_sems,
              recv_data_sems, recv_dest_sems):
    my = lax.axis_index("i")
    dvals = dest_ref[...]

    def slot_start(dst):
        soff = jnp.sum((dvals < dst).astype(jnp.int32))
        return jnp.minimum((soff // ALIGN) * ALIGN, T - SLOT)

    cp = pltpu.make_async_copy(
        xs_ref.at[pl.ds(slot_start(my), SLOT), :],
        stage_ref.at[my], local_sems.at[0])
    cp.start()
    cp2 = pltpu.make_async_copy(dest_ref, dall_ref.at[my], local_sems.at[1])
    cp2.start()

    data_rdmas = []
    dest_rdmas = []
    for d in range(1, N_DEV):
        tgt = lax.rem(my + d, N_DEV)
        r1 = pltpu.make_async_remote_copy(
            src_ref=xs_ref.at[pl.ds(slot_start(tgt), SLOT), :],
            dst_ref=stage_ref.at[my],
            send_sem=send_data_sems.at[d - 1],
            recv_sem=recv_data_sems.at[my],
            device_id=(tgt,),
            device_id_type=pl.DeviceIdType.MESH,
        )
        r1.start()
        r2 = pltpu.make_async_remote_copy(
            src_ref=dest_ref,
            dst_ref=dall_ref.at[my],
            send_sem=send_dest_sems.at[d - 1],
            recv_sem=recv_dest_sems.at[my],
            device_id=(tgt,),
            device_id_type=pl.DeviceIdType.MESH,
        )
        r2.start()
        data_rdmas.append(r1)
        dest_rdmas.append(r2)

    cp.wait()
    cp2.wait()
    for r in data_rdmas:
        r.wait_send()
    for r in dest_rdmas:
        r.wait_send()

    for d in range(1, N_DEV):
        src = lax.rem(my + d, N_DEV)
        pltpu.make_async_remote_copy(
            src_ref=stage_ref.at[src], dst_ref=stage_ref.at[src],
            send_sem=send_data_sems.at[d - 1],
            recv_sem=recv_data_sems.at[src],
            device_id=(my,), device_id_type=pl.DeviceIdType.MESH,
        ).wait_recv()
        pltpu.make_async_remote_copy(
            src_ref=dall_ref.at[src], dst_ref=dall_ref.at[src],
            send_sem=send_dest_sems.at[d - 1],
            recv_sem=recv_dest_sems.at[src],
            device_id=(my,), device_id_type=pl.DeviceIdType.MESH,
        ).wait_recv()


def kernel(x, dest):
    xbf = x.astype(jnp.bfloat16)
    dest = dest.astype(jnp.int32)
    order = jnp.argsort(dest, stable=True)
    xs = xbf[order]
    d2 = dest.reshape(ALIGN, 128)

    stage, dall = pl.pallas_call(
        _a2a_body,
        out_shape=[
            jax.ShapeDtypeStruct((N_DEV, SLOT, D), jnp.bfloat16),
            jax.ShapeDtypeStruct((N_DEV, ALIGN, 128), jnp.int32),
        ],
        in_specs=[pl.BlockSpec(memory_space=pltpu.VMEM),
                  pl.BlockSpec(memory_space=pltpu.VMEM)],
        out_specs=[pl.BlockSpec(memory_space=pltpu.VMEM),
                   pl.BlockSpec(memory_space=pltpu.VMEM)],
        scratch_shapes=[
            pltpu.SemaphoreType.DMA((2,)),
            pltpu.SemaphoreType.DMA((N_DEV - 1,)),
            pltpu.SemaphoreType.DMA((N_DEV - 1,)),
            pltpu.SemaphoreType.DMA((N_DEV,)),
            pltpu.SemaphoreType.DMA((N_DEV,)),
        ],
        compiler_params=pltpu.CompilerParams(collective_id=0),
    )(xs, d2)

    my = lax.axis_index("i")
    dall = dall.reshape(N_DEV, T)
    cnt = jnp.sum(dall == my, axis=1).astype(jnp.int32)
    roff = jnp.cumsum(cnt) - cnt
    soff = jnp.sum(dall < my, axis=1).astype(jnp.int32)
    start = jnp.minimum((soff // ALIGN) * ALIGN, T - SLOT)
    inslot = soff - start
    j = jnp.arange(T, dtype=jnp.int32)
    src = jnp.searchsorted(roff, j, side="right").astype(jnp.int32) - 1
    row = inslot[src] + j - roff[src]
    return stage[src, row]


# baseline (device time: 186970 ns/iter reference)
import jax
import jax.numpy as jnp
from jax import lax
from jax.experimental import pallas as pl
from jax.experimental.pallas import tpu as pltpu

N_DEV = 8
T = 2048
D = 1024
SLOT = 512
ALIGN = 16


def _a2a_body(xs_ref, dest_ref, stage_ref, dall_ref,
              local_sems, send_data_sems, send_dest_sems,
              recv_data_sems, recv_dest_sems):
    my = lax.axis_index("i")
    dvals = dest_ref[...]

    def slot_start(dst):
        soff = jnp.sum((dvals < dst).astype(jnp.int32))
        return jnp.minimum((soff // ALIGN) * ALIGN, T - SLOT)

    cp = pltpu.make_async_copy(
        xs_ref.at[pl.ds(slot_start(my), SLOT), :],
        stage_ref.at[my], local_sems.at[0])
    cp.start()
    cp2 = pltpu.make_async_copy(dest_ref, dall_ref.at[my], local_sems.at[1])
    cp2.start()

    data_rdmas = []
    dest_rdmas = []
    for d in range(1, N_DEV):
        tgt = lax.rem(my + d, N_DEV)
        r1 = pltpu.make_async_remote_copy(
            src_ref=xs_ref.at[pl.ds(slot_start(tgt), SLOT), :],
            dst_ref=stage_ref.at[my],
            send_sem=send_data_sems.at[d - 1],
            recv_sem=recv_data_sems.at[my],
            device_id=(tgt,),
            device_id_type=pl.DeviceIdType.MESH,
        )
        r1.start()
        r2 = pltpu.make_async_remote_copy(
            src_ref=dest_ref,
            dst_ref=dall_ref.at[my],
            send_sem=send_dest_sems.at[d - 1],
            recv_sem=recv_dest_sems.at[my],
            device_id=(tgt,),
            device_id_type=pl.DeviceIdType.MESH,
        )
        r2.start()
        data_rdmas.append(r1)
        dest_rdmas.append(r2)

    cp.wait()
    cp2.wait()
    for r in data_rdmas:
        r.wait_send()
    for r in dest_rdmas:
        r.wait_send()

    for d in range(1, N_DEV):
        src = lax.rem(my + d, N_DEV)
        pltpu.make_async_remote_copy(
            src_ref=stage_ref.at[src], dst_ref=stage_ref.at[src],
            send_sem=send_data_sems.at[d - 1],
            recv_sem=recv_data_sems.at[src],
            device_id=(my,), device_id_type=pl.DeviceIdType.MESH,
        ).wait_recv()
        pltpu.make_async_remote_copy(
            src_ref=dall_ref.at[src], dst_ref=dall_ref.at[src],
            send_sem=send_dest_sems.at[d - 1],
            recv_sem=recv_dest_sems.at[src],
            device_id=(my,), device_id_type=pl.DeviceIdType.MESH,
        ).wait_recv()


def kernel(x, dest):
    xbf = x.astype(jnp.bfloat16)
    dest = dest.astype(jnp.int32)
    order = jnp.argsort(dest, stable=True)
    xs = xbf[order]
    d2 = dest.reshape(ALIGN, 128)

    stage, dall = pl.pallas_call(
        _a2a_body,
        out_shape=[
            jax.ShapeDtypeStruct((N_DEV, SLOT, D), jnp.bfloat16),
            jax.ShapeDtypeStruct((N_DEV, ALIGN, 128), jnp.int32),
        ],
        in_specs=[pl.BlockSpec(memory_space=pltpu.VMEM),
                  pl.BlockSpec(memory_space=pltpu.VMEM)],
        out_specs=[pl.BlockSpec(memory_space=pltpu.VMEM),
                   pl.BlockSpec(memory_space=pltpu.VMEM)],
        scratch_shapes=[
            pltpu.SemaphoreType.DMA((2,)),
            pltpu.SemaphoreType.DMA((N_DEV - 1,)),
            pltpu.SemaphoreType.DMA((N_DEV - 1,)),
            pltpu.SemaphoreType.DMA((N_DEV,)),
            pltpu.SemaphoreType.DMA((N_DEV,)),
        ],
    )(xs, d2)

    my = lax.axis_index("i")
    dall = dall.reshape(N_DEV, T)
    cnt = jnp.sum(dall == my, axis=1).astype(jnp.int32)
    roff = jnp.cumsum(cnt) - cnt
    soff = jnp.sum(dall < my, axis=1).astype(jnp.int32)
    start = jnp.minimum((soff // ALIGN) * ALIGN, T - SLOT)
    inslot = soff - start
    j = jnp.arange(T, dtype=jnp.int32)
    src = jnp.searchsorted(roff, j, side="right").astype(jnp.int32) - 1
    row = inslot[src] + j - roff[src]
    return stage[src, row]


# device time: 99770 ns/iter; 1.8740x vs baseline; 1.8740x over previous
import jax
import jax.numpy as jnp
from jax import lax
from jax.experimental import pallas as pl
from jax.experimental.pallas import tpu as pltpu

N_DEV = 8
T = 2048
D = 1024
SLOT = 512
ALIGN = 16
CHUNK = 64


def _a2a_body(xs_ref, dest_ref, out_ref, stage_ref, dall_ref,
              local_sems, send_data_sems, send_dest_sems,
              recv_data_sems, recv_dest_sems, cp_sem):
    my = lax.axis_index("i")
    dvals = dest_ref[...]

    def slot_start(dst):
        soff = jnp.sum((dvals < dst).astype(jnp.int32))
        return jnp.minimum(soff, T - SLOT)

    cp = pltpu.make_async_copy(
        xs_ref.at[pl.ds(slot_start(my), SLOT)],
        stage_ref.at[my], local_sems.at[0])
    cp.start()
    cp2 = pltpu.make_async_copy(dest_ref, dall_ref.at[my], local_sems.at[1])
    cp2.start()

    data_rdmas = []
    dest_rdmas = []
    for d in range(1, N_DEV):
        tgt = lax.rem(my + d, N_DEV)
        r1 = pltpu.make_async_remote_copy(
            src_ref=xs_ref.at[pl.ds(slot_start(tgt), SLOT)],
            dst_ref=stage_ref.at[my],
            send_sem=send_data_sems.at[d - 1],
            recv_sem=recv_data_sems.at[my],
            device_id=(tgt,),
            device_id_type=pl.DeviceIdType.MESH,
        )
        r1.start()
        r2 = pltpu.make_async_remote_copy(
            src_ref=dest_ref,
            dst_ref=dall_ref.at[my],
            send_sem=send_dest_sems.at[d - 1],
            recv_sem=recv_dest_sems.at[my],
            device_id=(tgt,),
            device_id_type=pl.DeviceIdType.MESH,
        )
        r2.start()
        data_rdmas.append(r1)
        dest_rdmas.append(r2)

    cp.wait()
    cp2.wait()
    for r in data_rdmas:
        r.wait_send()
    for r in dest_rdmas:
        r.wait_send()

    for d in range(1, N_DEV):
        src = lax.rem(my + d, N_DEV)
        pltpu.make_async_remote_copy(
            src_ref=stage_ref.at[src], dst_ref=stage_ref.at[src],
            send_sem=send_data_sems.at[d - 1],
            recv_sem=recv_data_sems.at[src],
            device_id=(my,), device_id_type=pl.DeviceIdType.MESH,
        ).wait_recv()
        pltpu.make_async_remote_copy(
            src_ref=dall_ref.at[src], dst_ref=dall_ref.at[src],
            send_sem=send_dest_sems.at[d - 1],
            recv_sem=recv_dest_sems.at[src],
            device_id=(my,), device_id_type=pl.DeviceIdType.MESH,
        ).wait_recv()

    roff = jnp.int32(0)
    for s in range(N_DEV):
        dall_s = dall_ref[s]
        cnt_s = jnp.sum((dall_s == my).astype(jnp.int32))
        soff_s = jnp.sum((dall_s < my).astype(jnp.int32))
        inslot_s = soff_s - jnp.minimum(soff_s, T - SLOT)
        nch = (cnt_s + CHUNK - 1) // CHUNK
        roff_s = roff

        def copy_chunk(k, carry, s=s, cnt=cnt_s, inslot=inslot_s, base=roff_s):
            off = jnp.maximum(0, jnp.minimum(k * CHUNK, cnt - CHUNK))
            c = pltpu.make_async_copy(
                stage_ref.at[s, pl.ds(inslot + off, CHUNK)],
                out_ref.at[pl.ds(base + off, CHUNK)],
                cp_sem)
            c.start()
            c.wait()
            return carry

        lax.fori_loop(0, nch, copy_chunk, 0)
        roff = roff + cnt_s


def kernel(x, dest):
    dest = dest.astype(jnp.int32)
    order = jnp.argsort(dest, stable=True)
    xs = x[order].astype(jnp.bfloat16).reshape(T, 8, 128)
    d2 = dest.reshape(ALIGN, 128)

    out = pl.pallas_call(
        _a2a_body,
        out_shape=jax.ShapeDtypeStruct((T, 8, 128), jnp.bfloat16),
        in_specs=[pl.BlockSpec(memory_space=pltpu.VMEM),
                  pl.BlockSpec(memory_space=pltpu.VMEM)],
        out_specs=pl.BlockSpec(memory_space=pltpu.VMEM),
        scratch_shapes=[
            pltpu.VMEM((N_DEV, SLOT, 8, 128), jnp.bfloat16),
            pltpu.VMEM((N_DEV, ALIGN, 128), jnp.int32),
            pltpu.SemaphoreType.DMA((2,)),
            pltpu.SemaphoreType.DMA((N_DEV - 1,)),
            pltpu.SemaphoreType.DMA((N_DEV - 1,)),
            pltpu.SemaphoreType.DMA((N_DEV,)),
            pltpu.SemaphoreType.DMA((N_DEV,)),
            pltpu.SemaphoreType.DMA,
        ],
    )(xs, d2)
    return out.reshape(T, D)


# device time: 64496 ns/iter; 2.8989x vs baseline; 1.5469x over previous
import jax
import jax.numpy as jnp
from jax import lax
from jax.experimental import pallas as pl
from jax.experimental.pallas import tpu as pltpu

N_DEV = 8
T = 2048
D = 1024
DR = 16
CHUNK = 64


def _a2a_body(xs_ref, dest_ref, out_ref, dall_ref,
              local_sems, send_dest_sems, recv_dest_sems,
              send_data_sems, recv_data_sem, cp_sem):
    my = lax.axis_index("i")

    cp = pltpu.make_async_copy(dest_ref, dall_ref.at[my], local_sems.at[0])
    cp.start()
    dest_rdmas = []
    for d in range(1, N_DEV):
        tgt = lax.rem(my + d, N_DEV)
        r = pltpu.make_async_remote_copy(
            src_ref=dest_ref,
            dst_ref=dall_ref.at[my],
            send_sem=send_dest_sems.at[d - 1],
            recv_sem=recv_dest_sems.at[my],
            device_id=(tgt,),
            device_id_type=pl.DeviceIdType.MESH,
        )
        r.start()
        dest_rdmas.append(r)
    cp.wait()
    for r in dest_rdmas:
        r.wait_send()
    for d in range(1, N_DEV):
        src = lax.rem(my + d, N_DEV)
        pltpu.make_async_remote_copy(
            src_ref=dall_ref.at[src], dst_ref=dall_ref.at[src],
            send_sem=send_dest_sems.at[d - 1],
            recv_sem=recv_dest_sems.at[src],
            device_id=(my,), device_id_type=pl.DeviceIdType.MESH,
        ).wait_recv()

    dvals = dest_ref[...]
    dall = dall_ref[...]
    srcidx = lax.broadcasted_iota(jnp.int32, (N_DEV, DR, 128), 0)

    def pair(t):
        cnt = jnp.sum((dvals == t).astype(jnp.int32))
        soff = jnp.sum((dvals < t).astype(jnp.int32))
        roff = jnp.sum(((dall == t) & (srcidx < my)).astype(jnp.int32))
        return cnt, soff, roff

    for d in range(1, N_DEV):
        tgt = lax.rem(my + d, N_DEV)
        cnt, soff, roff = pair(tgt)

        def send_chunk(k, c, tgt=tgt, d=d, cnt=cnt, soff=soff, roff=roff):
            off = jnp.maximum(0, jnp.minimum(k * CHUNK, cnt - CHUNK))
            pltpu.make_async_remote_copy(
                src_ref=xs_ref.at[pl.ds(soff + off, CHUNK)],
                dst_ref=out_ref.at[pl.ds(roff + off, CHUNK)],
                send_sem=send_data_sems.at[d - 1],
                recv_sem=recv_data_sem,
                device_id=(tgt,),
                device_id_type=pl.DeviceIdType.MESH,
            ).start()
            return c

        lax.fori_loop(0, (cnt + CHUNK - 1) // CHUNK, send_chunk, 0)

    cnt_m, soff_m, roff_m = pair(my)

    def own_chunk(k, c):
        off = jnp.maximum(0, jnp.minimum(k * CHUNK, cnt_m - CHUNK))
        c2 = pltpu.make_async_copy(
            xs_ref.at[pl.ds(soff_m + off, CHUNK)],
            out_ref.at[pl.ds(roff_m + off, CHUNK)],
            cp_sem)
        c2.start()
        c2.wait()
        return c

    lax.fori_loop(0, (cnt_m + CHUNK - 1) // CHUNK, own_chunk, 0)

    for d in range(1, N_DEV):
        tgt = lax.rem(my + d, N_DEV)
        cnt, _, _ = pair(tgt)

        def wait_send_chunk(k, c, tgt=tgt, d=d):
            pltpu.make_async_remote_copy(
                src_ref=xs_ref.at[pl.ds(0, CHUNK)],
                dst_ref=out_ref.at[pl.ds(0, CHUNK)],
                send_sem=send_data_sems.at[d - 1],
                recv_sem=recv_data_sem,
                device_id=(tgt,),
                device_id_type=pl.DeviceIdType.MESH,
            ).wait_send()
            return c

        lax.fori_loop(0, (cnt + CHUNK - 1) // CHUNK, wait_send_chunk, 0)

    total_in = jnp.int32(0)
    for s in range(N_DEV):
        cnt_s = jnp.sum((dall[s] == my).astype(jnp.int32))
        nch_s = (cnt_s + CHUNK - 1) // CHUNK
        total_in = total_in + jnp.where(my == s, 0, nch_s)

    def wait_recv_chunk(k, c):
        pltpu.make_async_remote_copy(
            src_ref=xs_ref.at[pl.ds(0, CHUNK)],
            dst_ref=out_ref.at[pl.ds(0, CHUNK)],
            send_sem=send_data_sems.at[0],
            recv_sem=recv_data_sem,
            device_id=(my,),
            device_id_type=pl.DeviceIdType.MESH,
        ).wait_recv()
        return c

    lax.fori_loop(0, total_in, wait_recv_chunk, 0)


def kernel(x, dest):
    dest = dest.astype(jnp.int32)
    order = jnp.argsort(dest, stable=True)
    xs = x[order].astype(jnp.bfloat16).reshape(T, 8, 128)
    d2 = dest.reshape(DR, 128)

    out = pl.pallas_call(
        _a2a_body,
        out_shape=jax.ShapeDtypeStruct((T, 8, 128), jnp.bfloat16),
        in_specs=[pl.BlockSpec(memory_space=pltpu.VMEM),
                  pl.BlockSpec(memory_space=pltpu.VMEM)],
        out_specs=pl.BlockSpec(memory_space=pltpu.VMEM),
        scratch_shapes=[
            pltpu.VMEM((N_DEV, DR, 128), jnp.int32),
            pltpu.SemaphoreType.DMA((1,)),
            pltpu.SemaphoreType.DMA((N_DEV - 1,)),
            pltpu.SemaphoreType.DMA((N_DEV,)),
            pltpu.SemaphoreType.DMA((N_DEV - 1,)),
            pltpu.SemaphoreType.DMA,
            pltpu.SemaphoreType.DMA,
        ],
    )(xs, d2)
    return out.reshape(T, D)


# device time: 63052 ns/iter; 2.9653x vs baseline; 1.0229x over previous
import jax
import jax.numpy as jnp
from jax import lax
from jax.experimental import pallas as pl
from jax.experimental.pallas import tpu as pltpu

N_DEV = 8
T = 2048
D = 1024
DR = 16
CHUNK = 32


def _a2a_body(xs_ref, dest_ref, out_ref, dall_ref,
              local_sems, send_dest_sems, recv_dest_sems,
              send_data_sems, recv_data_sem, cp_sem):
    my = lax.axis_index("i")

    cp = pltpu.make_async_copy(dest_ref, dall_ref.at[my], local_sems.at[0])
    cp.start()
    dest_rdmas = []
    for d in range(1, N_DEV):
        tgt = lax.rem(my + d, N_DEV)
        r = pltpu.make_async_remote_copy(
            src_ref=dest_ref,
            dst_ref=dall_ref.at[my],
            send_sem=send_dest_sems.at[d - 1],
            recv_sem=recv_dest_sems.at[my],
            device_id=(tgt,),
            device_id_type=pl.DeviceIdType.MESH,
        )
        r.start()
        dest_rdmas.append(r)
    cp.wait()
    for r in dest_rdmas:
        r.wait_send()
    for d in range(1, N_DEV):
        src = lax.rem(my + d, N_DEV)
        pltpu.make_async_remote_copy(
            src_ref=dall_ref.at[src], dst_ref=dall_ref.at[src],
            send_sem=send_dest_sems.at[d - 1],
            recv_sem=recv_dest_sems.at[src],
            device_id=(my,), device_id_type=pl.DeviceIdType.MESH,
        ).wait_recv()

    dvals = dest_ref[...]
    dall = dall_ref[...]
    srcidx = lax.broadcasted_iota(jnp.int32, (N_DEV, DR, 128), 0)

    def pair(t):
        cnt = jnp.sum((dvals == t).astype(jnp.int32))
        soff = jnp.sum((dvals < t).astype(jnp.int32))
        roff = jnp.sum(((dall == t) & (srcidx < my)).astype(jnp.int32))
        return cnt, soff, roff

    for d in range(1, N_DEV):
        tgt = lax.rem(my + d, N_DEV)
        cnt, soff, roff = pair(tgt)

        def send_chunk(k, c, tgt=tgt, d=d, cnt=cnt, soff=soff, roff=roff):
            off = jnp.maximum(0, jnp.minimum(k * CHUNK, cnt - CHUNK))
            pltpu.make_async_remote_copy(
                src_ref=xs_ref.at[pl.ds(soff + off, CHUNK)],
                dst_ref=out_ref.at[pl.ds(roff + off, CHUNK)],
                send_sem=send_data_sems.at[d - 1],
                recv_sem=recv_data_sem,
                device_id=(tgt,),
                device_id_type=pl.DeviceIdType.MESH,
            ).start()
            return c

        lax.fori_loop(0, (cnt + CHUNK - 1) // CHUNK, send_chunk, 0)

    cnt_m, soff_m, roff_m = pair(my)

    def own_chunk(k, c):
        off = jnp.maximum(0, jnp.minimum(k * CHUNK, cnt_m - CHUNK))
        c2 = pltpu.make_async_copy(
            xs_ref.at[pl.ds(soff_m + off, CHUNK)],
            out_ref.at[pl.ds(roff_m + off, CHUNK)],
            cp_sem)
        c2.start()
        c2.wait()
        return c

    lax.fori_loop(0, (cnt_m + CHUNK - 1) // CHUNK, own_chunk, 0)

    for d in range(1, N_DEV):
        tgt = lax.rem(my + d, N_DEV)
        cnt, _, _ = pair(tgt)

        def wait_send_chunk(k, c, tgt=tgt, d=d):
            pltpu.make_async_remote_copy(
                src_ref=xs_ref.at[pl.ds(0, CHUNK)],
                dst_ref=out_ref.at[pl.ds(0, CHUNK)],
                send_sem=send_data_sems.at[d - 1],
                recv_sem=recv_data_sem,
                device_id=(tgt,),
                device_id_type=pl.DeviceIdType.MESH,
            ).wait_send()
            return c

        lax.fori_loop(0, (cnt + CHUNK - 1) // CHUNK, wait_send_chunk, 0)

    total_in = jnp.int32(0)
    for s in range(N_DEV):
        cnt_s = jnp.sum((dall[s] == my).astype(jnp.int32))
        nch_s = (cnt_s + CHUNK - 1) // CHUNK
        total_in = total_in + jnp.where(my == s, 0, nch_s)

    def wait_recv_chunk(k, c):
        pltpu.make_async_remote_copy(
            src_ref=xs_ref.at[pl.ds(0, CHUNK)],
            dst_ref=out_ref.at[pl.ds(0, CHUNK)],
            send_sem=send_data_sems.at[0],
            recv_sem=recv_data_sem,
            device_id=(my,),
            device_id_type=pl.DeviceIdType.MESH,
        ).wait_recv()
        return c

    lax.fori_loop(0, total_in, wait_recv_chunk, 0)


def kernel(x, dest):
    dest = dest.astype(jnp.int32)
    order = jnp.argsort(dest, stable=True)
    xs = x[order].astype(jnp.bfloat16).reshape(T, 8, 128)
    d2 = dest.reshape(DR, 128)

    out = pl.pallas_call(
        _a2a_body,
        out_shape=jax.ShapeDtypeStruct((T, 8, 128), jnp.bfloat16),
        in_specs=[pl.BlockSpec(memory_space=pltpu.VMEM),
                  pl.BlockSpec(memory_space=pltpu.VMEM)],
        out_specs=pl.BlockSpec(memory_space=pltpu.VMEM),
        scratch_shapes=[
            pltpu.VMEM((N_DEV, DR, 128), jnp.int32),
            pltpu.SemaphoreType.DMA((1,)),
            pltpu.SemaphoreType.DMA((N_DEV - 1,)),
            pltpu.SemaphoreType.DMA((N_DEV,)),
            pltpu.SemaphoreType.DMA((N_DEV - 1,)),
            pltpu.SemaphoreType.DMA,
            pltpu.SemaphoreType.DMA,
        ],
    )(xs, d2)
    return out.reshape(T, D)
